# baseline (speedup 1.0000x reference)
import functools

import jax
import jax.numpy as jnp
from jax import lax
from jax.experimental import pallas as pl
from jax.experimental.pallas import tpu as pltpu
from jax.experimental.pallas import tpu_sc as plsc

N = 10000
E = 320000
DF = 128
HD = 64
HEADS = 8
OC1 = 8

NC = 2
NS = 16
NW = NC * NS
EW = E // NW
B = 80
CH = 125
NP = 10240
NPT = NP // NS

f32 = jnp.float32
i32 = jnp.int32

_HIGH = lax.Precision.HIGHEST


def _dense1_body(x_ref, w1_ref, as_ref, ad_ref, h_ref, asrc_ref, adst_ref):
    h = jnp.dot(x_ref[...], w1_ref[...], preferred_element_type=f32,
                precision=_HIGH)
    h_ref[...] = h
    asrc_ref[...] = jnp.dot(h, as_ref[...], preferred_element_type=f32,
                            precision=_HIGH)
    adst_ref[...] = jnp.dot(h, ad_ref[...], preferred_element_type=f32,
                            precision=_HIGH)


def _dense1(x, w1, a_src_mat, a_dst_mat):
    return pl.pallas_call(
        _dense1_body,
        out_shape=[
            jax.ShapeDtypeStruct((N, HD), f32),
            jax.ShapeDtypeStruct((N, HEADS), f32),
            jax.ShapeDtypeStruct((N, HEADS), f32),
        ],
    )(x, w1, a_src_mat, a_dst_mat)


def _edge1_body(asrc_hbm, adst_hbm, h_hbm, src3_hbm, dst3_hbm, z64_hbm,
                z8_hbm,
                num_out, den_out,
                num_sh, den_sh, src2_v, dst2_v,
                as_v0, ad_v0, h_v0, w_v0, msg_v0,
                as_v1, ad_v1, h_v1, w_v1, msg_v1,
                sem_g0, sem_g1, sem_s0, sem_s1):
    sid = lax.axis_index("s")
    cid = lax.axis_index("c")
    wid = sid * NC + cid
    r0 = sid * NPT

    as_v = (as_v0, as_v1)
    ad_v = (ad_v0, ad_v1)
    h_v = (h_v0, h_v1)
    w_v = (w_v0, w_v1)
    msg_v = (msg_v0, msg_v1)
    sem_g = (sem_g0, sem_g1)
    sem_s = (sem_s0, sem_s1)

    pltpu.sync_copy(src3_hbm.at[wid], src2_v)
    pltpu.sync_copy(dst3_hbm.at[wid], dst2_v)

    pltpu.sync_copy(z64_hbm.at[pl.ds(r0, NPT)], num_sh.at[pl.ds(r0, NPT)])
    pltpu.sync_copy(z8_hbm.at[pl.ds(r0, NPT)], den_sh.at[pl.ds(r0, NPT)])
    plsc.subcore_barrier()

    io16 = lax.iota(i32, 16)
    half = io16 // 8
    col8 = io16 % 8

    def fire_gathers(c, s):
        pltpu.async_copy(asrc_hbm.at[src2_v.at[c]], as_v[s], sem_g[s])
        pltpu.async_copy(adst_hbm.at[dst2_v.at[c]], ad_v[s], sem_g[s])
        pltpu.async_copy(h_hbm.at[src2_v.at[c]], h_v[s], sem_g[s])

    def wait_gathers(s):
        pltpu.make_async_copy(asrc_hbm.at[src2_v.at[0]], as_v[s],
                              sem_g[s]).wait()
        pltpu.make_async_copy(adst_hbm.at[dst2_v.at[0]], ad_v[s],
                              sem_g[s]).wait()
        pltpu.make_async_copy(h_hbm.at[src2_v.at[0]], h_v[s],
                              sem_g[s]).wait()

    def fire_scatters(c, s):
        pltpu.async_copy(w_v[s], den_sh.at[dst2_v.at[c]], sem_s[s],
                         add=True)
        pltpu.async_copy(msg_v[s], num_sh.at[dst2_v.at[c]], sem_s[s],
                         add=True)

    def wait_scatters(s):
        pltpu.make_async_copy(w_v[s], den_sh.at[dst2_v.at[0]],
                              sem_s[s]).wait()
        pltpu.make_async_copy(msg_v[s], num_sh.at[dst2_v.at[0]],
                              sem_s[s]).wait()

    def compute(s):
        av, dv, hvr, wv, mv = as_v[s], ad_v[s], h_v[s], w_v[s], msg_v[s]

        def pair_body(e2, carry2):
            row = 2 * e2 + half
            x = (plsc.load_gather(av, [row, col8])
                 + plsc.load_gather(dv, [row, col8]))
            w16 = jnp.exp(jnp.maximum(x, 0.2 * x))
            plsc.store_scatter(wv, [row, col8], w16)
            for j in range(8):
                e = 2 * e2 + (j // 4)
                hvec = hvr[e, pl.ds((j % 4) * 16, 16)]
                bw = jnp.take_along_axis(w16, 2 * j + half, axis=0)
                mv[e, pl.ds((j % 4) * 16, 16)] = hvec * bw
            return carry2

        lax.fori_loop(0, B // 2, pair_body, 0, unroll=8)

    fire_gathers(0, 0)
    fire_gathers(1, 1)

    def pair_of_chunks(cc, carry):
        c0 = 2 * cc

        wait_gathers(0)

        @pl.when(cc != 0)
        def _():
            wait_scatters(0)

        compute(0)
        fire_scatters(c0, 0)
        fire_gathers(c0 + 2, 0)

        wait_gathers(1)

        @pl.when(cc != 0)
        def _():
            wait_scatters(1)

        compute(1)
        fire_scatters(c0 + 1, 1)

        @pl.when(c0 + 3 < CH)
        def _():
            fire_gathers(c0 + 3, 1)

        return carry

    lax.fori_loop(0, CH // 2, pair_of_chunks, 0)

    wait_gathers(0)
    wait_scatters(0)
    compute(0)
    fire_scatters(CH - 1, 0)
    wait_scatters(0)
    wait_scatters(1)

    plsc.subcore_barrier()

    pltpu.sync_copy(num_sh.at[pl.ds(r0, NPT)],
                    num_out.at[cid, pl.ds(r0, NPT)])
    pltpu.sync_copy(den_sh.at[pl.ds(r0, NPT)],
                    den_out.at[cid, pl.ds(r0, NPT)])


def _edge1(asrc, adst, h, src3, dst3, z64, z8):
    mesh = plsc.VectorSubcoreMesh(
        core_axis_name="c", subcore_axis_name="s",
        num_cores=NC, num_subcores=NS)
    buf = lambda: [
        pltpu.VMEM((B, HEADS), f32),
        pltpu.VMEM((B, HEADS), f32),
        pltpu.VMEM((B, HD), f32),
        pltpu.VMEM((B, HEADS), f32),
        pltpu.VMEM((B, HD), f32),
    ]
    fn = pl.kernel(
        _edge1_body,
        out_type=[
            jax.ShapeDtypeStruct((NC, NP, HD), f32),
            jax.ShapeDtypeStruct((NC, NP, HEADS), f32),
        ],
        mesh=mesh,
        compiler_params=pltpu.CompilerParams(needs_layout_passes=False, use_tc_tiling_on_sc=False),
        scratch_types=[
            pltpu.VMEM_SHARED((NP, HD), f32),
            pltpu.VMEM_SHARED((NP, HEADS), f32),
            pltpu.VMEM((CH, B), i32),
            pltpu.VMEM((CH, B), i32),
            *buf(),
            *buf(),
            pltpu.SemaphoreType.DMA,
            pltpu.SemaphoreType.DMA,
            pltpu.SemaphoreType.DMA,
            pltpu.SemaphoreType.DMA,
        ],
    )
    return fn(asrc, adst, h, src3, dst3, z64, z8)


def _combine_body(nump_ref, denp_ref, h_ref, asrc_ref, adst_ref, b1_ref,
                  k8_ref, w2_ref, sc2_ref, sd2_ref,
                  h2_ref, u_ref, v_ref):
    hmat = h_ref[...]
    al = asrc_ref[...] + adst_ref[...]
    wself = jnp.exp(jnp.maximum(al, 0.2 * al))
    den = denp_ref[0] + denp_ref[1] + wself
    wwide = jnp.dot(wself, k8_ref[...], preferred_element_type=f32,
                    precision=_HIGH)
    num = nump_ref[0] + nump_ref[1] + wwide * hmat
    denw = jnp.dot(den, k8_ref[...], preferred_element_type=f32,
                   precision=_HIGH) + 1e-16
    g = jnp.maximum(num / denw + b1_ref[...], 0.0)
    h2 = jnp.dot(g, w2_ref[...], preferred_element_type=f32,
                 precision=_HIGH)
    h2_ref[...] = h2
    u_ref[...] = h2 * sc2_ref[...]
    v_ref[...] = h2 * sd2_ref[...]


_CR = 1000


def _combine(num_p, den_p, h, asrc, adst, b1, k8, w2, a_src2, a_dst2):
    row = lambda i: (i, 0)
    full = lambda i: (0, 0)
    return pl.pallas_call(
        _combine_body,
        grid=(N // _CR,),
        in_specs=[
            pl.BlockSpec((2, _CR, HD), lambda i: (0, i, 0)),
            pl.BlockSpec((2, _CR, HEADS), lambda i: (0, i, 0)),
            pl.BlockSpec((_CR, HD), row),
            pl.BlockSpec((_CR, HEADS), row),
            pl.BlockSpec((_CR, HEADS), row),
            pl.BlockSpec((1, HD), full),
            pl.BlockSpec((HEADS, HD), full),
            pl.BlockSpec((HD, 1), full),
            pl.BlockSpec((1, 1), full),
            pl.BlockSpec((1, 1), full),
        ],
        out_specs=[
            pl.BlockSpec((_CR, 1), row),
            pl.BlockSpec((_CR, 1), row),
            pl.BlockSpec((_CR, 1), row),
        ],
        out_shape=[
            jax.ShapeDtypeStruct((N, 1), f32),
            jax.ShapeDtypeStruct((N, 1), f32),
            jax.ShapeDtypeStruct((N, 1), f32),
        ],
    )(num_p, den_p, h, asrc, adst, b1, k8, w2, a_src2, a_dst2)


def _edge2_body(u_hbm, v_hbm, h2_hbm, src_hbm, dst_hbm,
                num2_out, den2_out,
                u_v, v_v, h2_v, src_v, dst_v, num2_v, den2_v):
    sid = lax.axis_index("s")
    cid = lax.axis_index("c")
    wid = sid * NC + cid

    pltpu.sync_copy(u_hbm, u_v)
    pltpu.sync_copy(v_hbm, v_v)
    pltpu.sync_copy(h2_hbm, h2_v)
    pltpu.sync_copy(src_hbm.at[pl.ds(wid * EW, EW)], src_v)
    pltpu.sync_copy(dst_hbm.at[pl.ds(wid * EW, EW)], dst_v)

    zero16 = jnp.zeros((16,), f32)

    def zbody(i, carry):
        num2_v[pl.ds(i * 16, 16)] = zero16
        den2_v[pl.ds(i * 16, 16)] = zero16
        return carry

    lax.fori_loop(0, N // 16, zbody, 0)

    def ebody(i, carry):
        s16 = src_v[pl.ds(i * 16, 16)]
        d16 = dst_v[pl.ds(i * 16, 16)]
        us = plsc.load_gather(u_v, [s16])
        vd = plsc.load_gather(v_v, [d16])
        hs = plsc.load_gather(h2_v, [s16])
        al = us + vd
        w = jnp.exp(jnp.maximum(al, 0.2 * al))
        plsc.addupdate_scatter(num2_v, [d16], w * hs)
        plsc.addupdate_scatter(den2_v, [d16], w)
        return carry

    lax.fori_loop(0, EW // 16, ebody, 0)

    pltpu.sync_copy(num2_v, num2_out.at[pl.ds(wid * N, N)])
    pltpu.sync_copy(den2_v, den2_out.at[pl.ds(wid * N, N)])


def _edge2(u, v, h2, src, dst):
    mesh = plsc.VectorSubcoreMesh(
        core_axis_name="c", subcore_axis_name="s",
        num_cores=NC, num_subcores=NS)
    fn = pl.kernel(
        _edge2_body,
        out_type=[
            jax.ShapeDtypeStruct((NW * N,), f32),
            jax.ShapeDtypeStruct((NW * N,), f32),
        ],
        mesh=mesh,
        compiler_params=pltpu.CompilerParams(needs_layout_passes=False, use_tc_tiling_on_sc=False),
        scratch_types=[
            pltpu.VMEM((N,), f32),
            pltpu.VMEM((N,), f32),
            pltpu.VMEM((N,), f32),
            pltpu.VMEM((EW,), i32),
            pltpu.VMEM((EW,), i32),
            pltpu.VMEM((N,), f32),
            pltpu.VMEM((N,), f32),
        ],
    )
    return fn(u, v, h2, src, dst)


def _final_body(nump_ref, denp_ref, u_ref, v_ref, h2_ref, b2_ref, out_ref):
    al = u_ref[...] + v_ref[...]
    ws = jnp.exp(jnp.maximum(al, 0.2 * al))
    nu = jnp.sum(nump_ref[...], axis=0, keepdims=True) + ws * h2_ref[...]
    de = jnp.sum(denp_ref[...], axis=0, keepdims=True) + ws + 1e-16
    r = nu / de
    out_ref[...] = jnp.sum(r, axis=1, keepdims=True) / N + b2_ref[...]


def _final(num2_p, den2_p, u_t, v_t, h2_t, b2):
    return pl.pallas_call(
        _final_body,
        out_shape=jax.ShapeDtypeStruct((1, 1), f32),
    )(num2_p, den2_p, u_t, v_t, h2_t, b2)


def kernel(x, edge_index, W1, a_src1, a_dst1, b1, W2, a_src2, a_dst2, b2):
    ei = edge_index.astype(i32)
    src = ei[0]
    dst = ei[1]

    rows = jnp.arange(HD, dtype=i32)
    a_src_mat = jnp.zeros((HD, HEADS), f32).at[rows, rows // OC1].set(
        a_src1.reshape(HD))
    a_dst_mat = jnp.zeros((HD, HEADS), f32).at[rows, rows // OC1].set(
        a_dst1.reshape(HD))
    k8 = jnp.kron(jnp.eye(HEADS, dtype=f32), jnp.ones((1, OC1), f32))

    z64 = jnp.zeros((NP, HD), f32)
    z8 = jnp.zeros((NP, HEADS), f32)

    h, asrc, adst = _dense1(x.astype(f32), W1, a_src_mat, a_dst_mat)
    num_p, den_p = _edge1(asrc, adst, h, src.reshape(NW, CH, B),
                          dst.reshape(NW, CH, B), z64, z8)
    h2, u, v = _combine(num_p, den_p, h, asrc, adst, b1.reshape(1, HD),
                        k8, W2, a_src2.reshape(1, 1), a_dst2.reshape(1, 1))
    num2_p, den2_p = _edge2(u.reshape(N), v.reshape(N), h2.reshape(N),
                            src, dst)
    out = _final(num2_p.reshape(NW, N), den2_p.reshape(NW, N),
                 u.reshape(1, N), v.reshape(1, N),
                 h2.reshape(1, N), b2.reshape(1, 1))
    return out


# speedup vs baseline: 1.0956x; 1.0956x over previous
import functools

import jax
import jax.numpy as jnp
from jax import lax
from jax.experimental import pallas as pl
from jax.experimental.pallas import tpu as pltpu
from jax.experimental.pallas import tpu_sc as plsc

N = 10000
E = 320000
DF = 128
HD = 64
HEADS = 8
OC1 = 8

NC = 2
NS = 16
NW = NC * NS
EW = E // NW
B = 80
CH = 125
NP = 10240
NPT = NP // NS

f32 = jnp.float32
i32 = jnp.int32

_HIGH = lax.Precision.DEFAULT


def _dense1_body(x_ref, w1_ref, as_ref, ad_ref, h_ref, asrc_ref, adst_ref):
    h = jnp.dot(x_ref[...], w1_ref[...], preferred_element_type=f32,
                precision=_HIGH)
    h_ref[...] = h
    asrc_ref[...] = jnp.dot(h, as_ref[...], preferred_element_type=f32,
                            precision=_HIGH)
    adst_ref[...] = jnp.dot(h, ad_ref[...], preferred_element_type=f32,
                            precision=_HIGH)


def _dense1(x, w1, a_src_mat, a_dst_mat):
    return pl.pallas_call(
        _dense1_body,
        out_shape=[
            jax.ShapeDtypeStruct((N, HD), f32),
            jax.ShapeDtypeStruct((N, HEADS), f32),
            jax.ShapeDtypeStruct((N, HEADS), f32),
        ],
    )(x, w1, a_src_mat, a_dst_mat)


def _edge1_body(asrc_hbm, adst_hbm, h_hbm, src3_hbm, dst3_hbm, z64_hbm,
                z8_hbm,
                num_out, den_out,
                num_sh, den_sh, src2_v, dst2_v,
                as_v0, ad_v0, h_v0, w_v0, msg_v0,
                as_v1, ad_v1, h_v1, w_v1, msg_v1,
                sem_g0, sem_g1, sem_s0, sem_s1):
    sid = lax.axis_index("s")
    cid = lax.axis_index("c")
    wid = sid * NC + cid
    r0 = sid * NPT

    as_v = (as_v0, as_v1)
    ad_v = (ad_v0, ad_v1)
    h_v = (h_v0, h_v1)
    w_v = (w_v0, w_v1)
    msg_v = (msg_v0, msg_v1)
    sem_g = (sem_g0, sem_g1)
    sem_s = (sem_s0, sem_s1)

    pltpu.sync_copy(src3_hbm.at[wid], src2_v)
    pltpu.sync_copy(dst3_hbm.at[wid], dst2_v)

    pltpu.sync_copy(z64_hbm.at[pl.ds(r0, NPT)], num_sh.at[pl.ds(r0, NPT)])
    pltpu.sync_copy(z8_hbm.at[pl.ds(r0, NPT)], den_sh.at[pl.ds(r0, NPT)])
    plsc.subcore_barrier()

    io16 = lax.iota(i32, 16)
    half = io16 // 8
    col8 = io16 % 8

    def fire_gathers(c, s):
        pltpu.async_copy(asrc_hbm.at[src2_v.at[c]], as_v[s], sem_g[s])
        pltpu.async_copy(adst_hbm.at[dst2_v.at[c]], ad_v[s], sem_g[s])
        pltpu.async_copy(h_hbm.at[src2_v.at[c]], h_v[s], sem_g[s])

    def wait_gathers(s):
        pltpu.make_async_copy(asrc_hbm.at[src2_v.at[0]], as_v[s],
                              sem_g[s]).wait()
        pltpu.make_async_copy(adst_hbm.at[dst2_v.at[0]], ad_v[s],
                              sem_g[s]).wait()
        pltpu.make_async_copy(h_hbm.at[src2_v.at[0]], h_v[s],
                              sem_g[s]).wait()

    def fire_scatters(c, s):
        pltpu.async_copy(w_v[s], den_sh.at[dst2_v.at[c]], sem_s[s],
                         add=True)
        pltpu.async_copy(msg_v[s], num_sh.at[dst2_v.at[c]], sem_s[s],
                         add=True)

    def wait_scatters(s):
        pltpu.make_async_copy(w_v[s], den_sh.at[dst2_v.at[0]],
                              sem_s[s]).wait()
        pltpu.make_async_copy(msg_v[s], num_sh.at[dst2_v.at[0]],
                              sem_s[s]).wait()

    def compute(s):
        av, dv, hvr, wv, mv = as_v[s], ad_v[s], h_v[s], w_v[s], msg_v[s]

        def pair_body(e2, carry2):
            row = 2 * e2 + half
            x = (plsc.load_gather(av, [row, col8])
                 + plsc.load_gather(dv, [row, col8]))
            w16 = jnp.exp(jnp.maximum(x, 0.2 * x))
            plsc.store_scatter(wv, [row, col8], w16)
            for j in range(8):
                e = 2 * e2 + (j // 4)
                hvec = hvr[e, pl.ds((j % 4) * 16, 16)]
                bw = jnp.take_along_axis(w16, 2 * j + half, axis=0)
                mv[e, pl.ds((j % 4) * 16, 16)] = hvec * bw
            return carry2

        lax.fori_loop(0, B // 2, pair_body, 0, unroll=2)

    fire_gathers(0, 0)
    fire_gathers(1, 1)

    def pair_of_chunks(cc, carry):
        c0 = 2 * cc

        wait_gathers(0)

        @pl.when(cc != 0)
        def _():
            wait_scatters(0)

        compute(0)
        fire_scatters(c0, 0)
        fire_gathers(c0 + 2, 0)

        wait_gathers(1)

        @pl.when(cc != 0)
        def _():
            wait_scatters(1)

        compute(1)
        fire_scatters(c0 + 1, 1)

        @pl.when(c0 + 3 < CH)
        def _():
            fire_gathers(c0 + 3, 1)

        return carry

    lax.fori_loop(0, CH // 2, pair_of_chunks, 0)

    wait_gathers(0)
    wait_scatters(0)
    compute(0)
    fire_scatters(CH - 1, 0)
    wait_scatters(0)
    wait_scatters(1)

    plsc.subcore_barrier()

    pltpu.sync_copy(num_sh.at[pl.ds(r0, NPT)],
                    num_out.at[cid, pl.ds(r0, NPT)])
    pltpu.sync_copy(den_sh.at[pl.ds(r0, NPT)],
                    den_out.at[cid, pl.ds(r0, NPT)])


def _edge1(asrc, adst, h, src3, dst3, z64, z8):
    mesh = plsc.VectorSubcoreMesh(
        core_axis_name="c", subcore_axis_name="s",
        num_cores=NC, num_subcores=NS)
    buf = lambda: [
        pltpu.VMEM((B, HEADS), f32),
        pltpu.VMEM((B, HEADS), f32),
        pltpu.VMEM((B, HD), f32),
        pltpu.VMEM((B, HEADS), f32),
        pltpu.VMEM((B, HD), f32),
    ]
    fn = pl.kernel(
        _edge1_body,
        out_type=[
            jax.ShapeDtypeStruct((NC, NP, HD), f32),
            jax.ShapeDtypeStruct((NC, NP, HEADS), f32),
        ],
        mesh=mesh,
        compiler_params=pltpu.CompilerParams(needs_layout_passes=False, use_tc_tiling_on_sc=False),
        scratch_types=[
            pltpu.VMEM_SHARED((NP, HD), f32),
            pltpu.VMEM_SHARED((NP, HEADS), f32),
            pltpu.VMEM((CH, B), i32),
            pltpu.VMEM((CH, B), i32),
            *buf(),
            *buf(),
            pltpu.SemaphoreType.DMA,
            pltpu.SemaphoreType.DMA,
            pltpu.SemaphoreType.DMA,
            pltpu.SemaphoreType.DMA,
        ],
    )
    return fn(asrc, adst, h, src3, dst3, z64, z8)


def _combine_body(nump_ref, denp_ref, h_ref, asrc_ref, adst_ref, b1_ref,
                  k8_ref, w2_ref, sc2_ref, sd2_ref,
                  h2_ref, u_ref, v_ref):
    hmat = h_ref[...]
    al = asrc_ref[...] + adst_ref[...]
    wself = jnp.exp(jnp.maximum(al, 0.2 * al))
    den = denp_ref[0] + denp_ref[1] + wself
    wwide = jnp.dot(wself, k8_ref[...], preferred_element_type=f32,
                    precision=_HIGH)
    num = nump_ref[0] + nump_ref[1] + wwide * hmat
    denw = jnp.dot(den, k8_ref[...], preferred_element_type=f32,
                   precision=_HIGH) + 1e-16
    g = jnp.maximum(num / denw + b1_ref[...], 0.0)
    h2 = jnp.dot(g, w2_ref[...], preferred_element_type=f32,
                 precision=_HIGH)
    h2_ref[...] = h2
    u_ref[...] = h2 * sc2_ref[...]
    v_ref[...] = h2 * sd2_ref[...]


_CR = 1000


def _combine(num_p, den_p, h, asrc, adst, b1, k8, w2, a_src2, a_dst2):
    row = lambda i: (i, 0)
    full = lambda i: (0, 0)
    return pl.pallas_call(
        _combine_body,
        grid=(N // _CR,),
        in_specs=[
            pl.BlockSpec((2, _CR, HD), lambda i: (0, i, 0)),
            pl.BlockSpec((2, _CR, HEADS), lambda i: (0, i, 0)),
            pl.BlockSpec((_CR, HD), row),
            pl.BlockSpec((_CR, HEADS), row),
            pl.BlockSpec((_CR, HEADS), row),
            pl.BlockSpec((1, HD), full),
            pl.BlockSpec((HEADS, HD), full),
            pl.BlockSpec((HD, 1), full),
            pl.BlockSpec((1, 1), full),
            pl.BlockSpec((1, 1), full),
        ],
        out_specs=[
            pl.BlockSpec((_CR, 1), row),
            pl.BlockSpec((_CR, 1), row),
            pl.BlockSpec((_CR, 1), row),
        ],
        out_shape=[
            jax.ShapeDtypeStruct((N, 1), f32),
            jax.ShapeDtypeStruct((N, 1), f32),
            jax.ShapeDtypeStruct((N, 1), f32),
        ],
    )(num_p, den_p, h, asrc, adst, b1, k8, w2, a_src2, a_dst2)


def _edge2_body(u_hbm, v_hbm, h2_hbm, src_hbm, dst_hbm,
                num2_out, den2_out,
                u_v, v_v, h2_v, src_v, dst_v, num2_v, den2_v):
    sid = lax.axis_index("s")
    cid = lax.axis_index("c")
    wid = sid * NC + cid

    pltpu.sync_copy(u_hbm, u_v)
    pltpu.sync_copy(v_hbm, v_v)
    pltpu.sync_copy(h2_hbm, h2_v)
    pltpu.sync_copy(src_hbm.at[pl.ds(wid * EW, EW)], src_v)
    pltpu.sync_copy(dst_hbm.at[pl.ds(wid * EW, EW)], dst_v)

    zero16 = jnp.zeros((16,), f32)

    def zbody(i, carry):
        num2_v[pl.ds(i * 16, 16)] = zero16
        den2_v[pl.ds(i * 16, 16)] = zero16
        return carry

    lax.fori_loop(0, N // 16, zbody, 0)

    def ebody(i, carry):
        s16 = src_v[pl.ds(i * 16, 16)]
        d16 = dst_v[pl.ds(i * 16, 16)]
        us = plsc.load_gather(u_v, [s16])
        vd = plsc.load_gather(v_v, [d16])
        hs = plsc.load_gather(h2_v, [s16])
        al = us + vd
        w = jnp.exp(jnp.maximum(al, 0.2 * al))
        plsc.addupdate_scatter(num2_v, [d16], w * hs)
        plsc.addupdate_scatter(den2_v, [d16], w)
        return carry

    lax.fori_loop(0, EW // 16, ebody, 0)

    pltpu.sync_copy(num2_v, num2_out.at[pl.ds(wid * N, N)])
    pltpu.sync_copy(den2_v, den2_out.at[pl.ds(wid * N, N)])


def _edge2(u, v, h2, src, dst):
    mesh = plsc.VectorSubcoreMesh(
        core_axis_name="c", subcore_axis_name="s",
        num_cores=NC, num_subcores=NS)
    fn = pl.kernel(
        _edge2_body,
        out_type=[
            jax.ShapeDtypeStruct((NW * N,), f32),
            jax.ShapeDtypeStruct((NW * N,), f32),
        ],
        mesh=mesh,
        compiler_params=pltpu.CompilerParams(needs_layout_passes=False, use_tc_tiling_on_sc=False),
        scratch_types=[
            pltpu.VMEM((N,), f32),
            pltpu.VMEM((N,), f32),
            pltpu.VMEM((N,), f32),
            pltpu.VMEM((EW,), i32),
            pltpu.VMEM((EW,), i32),
            pltpu.VMEM((N,), f32),
            pltpu.VMEM((N,), f32),
        ],
    )
    return fn(u, v, h2, src, dst)


def _final_body(nump_ref, denp_ref, u_ref, v_ref, h2_ref, b2_ref, out_ref):
    al = u_ref[...] + v_ref[...]
    ws = jnp.exp(jnp.maximum(al, 0.2 * al))
    nu = jnp.sum(nump_ref[...], axis=0, keepdims=True) + ws * h2_ref[...]
    de = jnp.sum(denp_ref[...], axis=0, keepdims=True) + ws + 1e-16
    r = nu / de
    out_ref[...] = jnp.sum(r, axis=1, keepdims=True) / N + b2_ref[...]


def _final(num2_p, den2_p, u_t, v_t, h2_t, b2):
    return pl.pallas_call(
        _final_body,
        out_shape=jax.ShapeDtypeStruct((1, 1), f32),
    )(num2_p, den2_p, u_t, v_t, h2_t, b2)


def kernel(x, edge_index, W1, a_src1, a_dst1, b1, W2, a_src2, a_dst2, b2):
    ei = edge_index.astype(i32)
    src = ei[0]
    dst = ei[1]

    rows = jnp.arange(HD, dtype=i32)
    a_src_mat = jnp.zeros((HD, HEADS), f32).at[rows, rows // OC1].set(
        a_src1.reshape(HD))
    a_dst_mat = jnp.zeros((HD, HEADS), f32).at[rows, rows // OC1].set(
        a_dst1.reshape(HD))
    k8 = jnp.kron(jnp.eye(HEADS, dtype=f32), jnp.ones((1, OC1), f32))

    z64 = jnp.zeros((NP, HD), f32)
    z8 = jnp.zeros((NP, HEADS), f32)

    h, asrc, adst = _dense1(x.astype(f32), W1, a_src_mat, a_dst_mat)
    num_p, den_p = _edge1(asrc, adst, h, src.reshape(NW, CH, B),
                          dst.reshape(NW, CH, B), z64, z8)
    h2, u, v = _combine(num_p, den_p, h, asrc, adst, b1.reshape(1, HD),
                        k8, W2, a_src2.reshape(1, 1), a_dst2.reshape(1, 1))
    num2_p, den2_p = _edge2(u.reshape(N), v.reshape(N), h2.reshape(N),
                            src, dst)
    out = _final(num2_p.reshape(NW, N), den2_p.reshape(NW, N),
                 u.reshape(1, N), v.reshape(1, N),
                 h2.reshape(1, N), b2.reshape(1, 1))
    return out
